# Initial kernel scaffold; baseline (speedup 1.0000x reference)
import jax
import jax.numpy as jnp
from jax.experimental import pallas as pl


def kernel(pos, norm, params, batch):
    raise NotImplementedError("write your pallas kernel here")


# speedup vs baseline: 12.5274x; 12.5274x over previous
import jax
import jax.numpy as jnp
from jax import lax
from jax.experimental import pallas as pl
from jax.experimental.pallas import tpu as pltpu
from jax.experimental.pallas import tpu_sc as plsc

N_FIXED = 4096
K_NB = 16
K_INTE = 3
NW = 32


def _sc_gather_multi(tables, idx):
    B = idx.shape[0]
    assert B % NW == 0, B
    bpw = B // NW
    CH = bpw if bpw <= 128 else 128
    assert bpw % CH == 0 and CH % 8 == 0, (B, bpw, CH)
    nch = bpw // CH
    nt = len(tables)
    Ds = [int(t.shape[1]) for t in tables]

    mesh = plsc.VectorSubcoreMesh(core_axis_name="c", subcore_axis_name="s")
    out_type = [jax.ShapeDtypeStruct((B, D), jnp.float32) for D in Ds]
    scratch = (
        [pltpu.VMEM((CH,), jnp.int32)]
        + [pltpu.VMEM((CH, D), jnp.float32) for D in Ds]
        + [pltpu.SemaphoreType.DMA]
    )

    def body(idx_hbm, *rest):
        tabs = rest[:nt]
        outs = rest[nt : 2 * nt]
        idx_v = rest[2 * nt]
        rows = rest[2 * nt + 1 : 2 * nt + 1 + nt]
        sem = rest[-1]
        wid = lax.axis_index("s") * 2 + lax.axis_index("c")
        base0 = wid * bpw

        def chunk(c, carry):
            base = base0 + c * CH
            pltpu.sync_copy(idx_hbm.at[pl.ds(base, CH)], idx_v)
            cps = [pltpu.async_copy(tabs[t].at[idx_v], rows[t], sem)
                   for t in range(nt)]
            for cp in cps:
                cp.wait()
            for t in range(nt):
                pltpu.sync_copy(rows[t], outs[t].at[pl.ds(base, CH)])
            return carry

        lax.fori_loop(0, nch, chunk, 0)

    fn = pl.kernel(
        body, out_type=out_type, mesh=mesh, scratch_types=scratch,
        compiler_params=pltpu.CompilerParams(use_tc_tiling_on_sc=False))
    return fn(idx, *tables)


def _topk_pallas(dmat, k):
    b, na, nb = dmat.shape
    R = min(na, 256)

    def body(d_ref, idx_ref, dist_ref):
        i_b = pl.program_id(0)
        d = d_ref[0]
        cols = lax.broadcasted_iota(jnp.int32, (R, nb), 1)
        idxs = []
        vals = []
        for _ in range(k):
            m = jnp.min(d, axis=1, keepdims=True)
            am = jnp.min(jnp.where(d == m, cols, nb), axis=1, keepdims=True)
            idxs.append(am)
            vals.append(m)
            d = jnp.where(cols == am, jnp.float32(jnp.inf), d)
        idx_ref[0] = jnp.concatenate(idxs, axis=1) + i_b * nb
        dist_ref[0] = jnp.maximum(jnp.concatenate(vals, axis=1), 0.0)

    return pl.pallas_call(
        body,
        grid=(b, na // R),
        in_specs=[pl.BlockSpec((1, R, nb), lambda i, j: (i, j, 0))],
        out_specs=[
            pl.BlockSpec((1, R, k), lambda i, j: (i, j, 0)),
            pl.BlockSpec((1, R, k), lambda i, j: (i, j, 0)),
        ],
        out_shape=[
            jax.ShapeDtypeStruct((b, na, k), jnp.int32),
            jax.ShapeDtypeStruct((b, na, k), jnp.float32),
        ],
    )(dmat)


def _knn_flat(p, k):
    sq = jnp.sum(p * p, -1)
    d = sq[:, :, None] + sq[:, None, :] - 2.0 * jnp.einsum('bnd,bmd->bnm', p, p)
    idx, _ = _topk_pallas(d, k)
    return idx


def _knn2_flat(pa, pb, k):
    sqa = jnp.sum(pa * pa, -1)
    sqb = jnp.sum(pb * pb, -1)
    d = sqa[:, :, None] + sqb[:, None, :] - 2.0 * jnp.einsum('bnd,bmd->bnm',
                                                             pa, pb)
    idx, dist = _topk_pallas(d, k)
    return dist, idx


def _norm_rows(x):
    axes = tuple(range(x.ndim - 1))
    m = jnp.mean(x, axes, keepdims=True)
    v = jnp.var(x, axes, keepdims=True)
    return (x - m) / jnp.sqrt(v + 1e-5)


def _lin(p, x):
    return x @ p['W'] + p['b']


def _mlp_seq(ps, x, last_acti=True, last_norm=True):
    n = len(ps)
    for i, p in enumerate(ps):
        x = _lin(p, x)
        last = (i == n - 1)
        if (not last) or last_norm:
            x = _norm_rows(x)
        if (not last) or last_acti:
            x = jax.nn.relu(x)
    return x


def _smlp(ps, x):
    for i, p in enumerate(ps):
        x = _lin(p, x)
        if i < len(ps) - 1:
            x = jax.nn.relu(x)
    return x


def _gather_rows(x, idxf):
    b, n, c = x.shape
    tab = x.reshape(b * n, c)
    pad = (-c) % 16
    if pad:
        tab = jnp.pad(tab, ((0, 0), (0, pad)))
    (g,) = _sc_gather_multi([tab], idxf.reshape(-1).astype(jnp.int32))
    g = g.reshape(idxf.shape + (c + pad,))
    if pad:
        g = g[..., :c]
    return g


def _gather_rows2(xa, xb, idxf):
    b, n, c = xa.shape
    ga, gb = _sc_gather_multi(
        [xa.reshape(b * n, c), xb.reshape(b * n, xb.shape[-1])],
        idxf.reshape(-1).astype(jnp.int32))
    return (ga.reshape(idxf.shape + (c,)),
            gb.reshape(idxf.shape + (xb.shape[-1],)))


def _ptb(p, x, pos, idxf, pj):
    q = _lin(p['Wq'], x)
    k = _lin(p['Wk'], x)
    v = _lin(p['Wv'], x)
    kj, vj = _gather_rows2(k, v, idxf)
    delta = _smlp(p['pos'], pos[:, :, None, :] - pj)
    a = _smlp(p['gamma'], q[:, :, None, :] - kj + delta)
    a = jax.nn.softmax(a, axis=2)
    agg = jnp.sum(a * (vj + delta), axis=2)
    return x + _lin(p['out'], agg)


def _td(p, x, pos, idxf, ratio=0.25):
    n = x.shape[1]
    stride = int(round(1.0 / ratio))
    m = n // stride
    samp = jnp.arange(m) * stride
    h = _mlp_seq(p['mlp'], x)
    hn = _gather_rows(h, idxf[:, samp])
    return jnp.max(hn, axis=2), pos[:, samp]


def _tu(p, xc, xs, pc, pf, k_inte):
    dist, idxf = _knn2_flat(pf, pc, k_inte)
    w = 1.0 / (dist + 1e-8)
    w = w / jnp.sum(w, -1, keepdims=True)
    fj = _gather_rows(_mlp_seq(p['up'], xc), idxf)
    interp = jnp.sum(w[..., None] * fj, axis=2)
    return interp + _mlp_seq(p['skip'], xs)


def kernel(pos, norm, params, batch):
    b = batch.shape[0] // N_FIXED
    n = pos.shape[0] // b
    K = K_NB
    p1 = pos.reshape(b, n, 3)
    x1 = jnp.concatenate([pos, norm], -1).reshape(b, n, 6)
    id1 = _knn_flat(p1, K)
    pj1 = _gather_rows(p1, id1)
    x1 = _mlp_seq(params['mlp1'], x1)
    for pt in params['ptbs1']:
        x1 = _ptb(pt, x1, p1, id1, pj1)
    x2, p2 = _td(params['td1'], x1, p1, id1)
    id2 = _knn_flat(p2, K)
    pj2 = _gather_rows(p2, id2)
    for pt in params['ptbs2']:
        x2 = _ptb(pt, x2, p2, id2, pj2)
    x3, p3 = _td(params['td2'], x2, p2, id2)
    id3 = _knn_flat(p3, K)
    pj3 = _gather_rows(p3, id3)
    for pt in params['ptbs3']:
        x3 = _ptb(pt, x3, p3, id3, pj3)
    x4, p4 = _td(params['td3'], x3, p3, id3)
    id4 = _knn_flat(p4, K)
    pj4 = _gather_rows(p4, id4)
    for pt in params['ptbs4']:
        x4 = _ptb(pt, x4, p4, id4, pj4)
    x5, p5 = _td(params['td4'], x4, p4, id4)
    id5 = _knn_flat(p5, min(K, p5.shape[1]))
    pj5 = _gather_rows(p5, id5)
    for pt in params['ptbs5']:
        x5 = _ptb(pt, x5, p5, id5, pj5)
    x_mean = jnp.mean(x5, axis=1)
    x_mean = _mlp_seq(params['mlp2'], x_mean)
    x_mean = jnp.broadcast_to(x_mean[:, None, :],
                              (b, x5.shape[1], x_mean.shape[-1]))
    x6 = jnp.concatenate([x5, x_mean], -1)
    x6 = _mlp_seq(params['mlp3'], x6)
    for pt in params['ptbs6']:
        x6 = _ptb(pt, x6, p5, id5, pj5)
    x7 = _tu(params['tu1'], x6, x4, p5, p4, K_INTE)
    for pt in params['ptbs7']:
        x7 = _ptb(pt, x7, p4, id4, pj4)
    x8 = _tu(params['tu2'], x7, x3, p4, p3, K_INTE)
    for pt in params['ptbs8']:
        x8 = _ptb(pt, x8, p3, id3, pj3)
    x9 = _tu(params['tu3'], x8, x2, p3, p2, K_INTE)
    for pt in params['ptbs9']:
        x9 = _ptb(pt, x9, p2, id2, pj2)
    x10 = _tu(params['tu4'], x9, x1, p2, p1, K_INTE)
    for pt in params['ptbs10']:
        x10 = _ptb(pt, x10, p1, id1, pj1)
    out = _mlp_seq(params['fc'], x10, last_acti=False, last_norm=False)
    return jnp.transpose(out, (0, 2, 1))
